# baseline (speedup 1.0000x reference)
import jax
import jax.numpy as jnp
from jax import lax
from jax.experimental import pallas as pl
from jax.experimental.pallas import tpu as pltpu
from jax.experimental.pallas import tpu_sc as plsc


def _gather_body(t5, idx5, out5, plane_v, idx_v, out_v):
    c = lax.axis_index("c")
    s = lax.axis_index("s")
    wid = s * 2 + c
    heavy = wid < 16
    b = lax.select(heavy, wid, wid - 16)
    bt = b // 8
    rb = b % 8
    pltpu.sync_copy(idx5.at[bt, :, rb, :], idx_v)

    def do_plane(p):
        with jax.named_scope("plane_dma"):
            pltpu.sync_copy(t5.at[p, bt, :, rb, :], plane_v)

        with jax.named_scope("gather"):
            @plsc.parallel_loop(0, 1024, step=1, unroll=8)
            def _(k):
                r = lax.shift_right_logical(k, 3)
                o = lax.bitwise_and(k, 7) * 16
                n = idx_v[r, pl.ds(o, 16)]
                hi = lax.shift_right_logical(n, 7)
                lo = lax.bitwise_and(n, 127)
                out_v[r, pl.ds(o, 16)] = plsc.load_gather(plane_v, [hi, lo])

        with jax.named_scope("out_dma"):
            pltpu.sync_copy(out_v, out5.at[p, bt, :, rb, :])

    do_plane(lax.select(heavy, 0, 1))

    @pl.when(heavy)
    def _():
        do_plane(2)


def kernel(t_in, t_idx):
    t5 = t_in.transpose(2, 0, 1).reshape(3, 2, 8, 512, 128).transpose(0, 1, 3, 2, 4)
    idx5 = t_idx.astype(jnp.int32).reshape(2, 8, 128, 128).transpose(0, 2, 1, 3)
    mesh = plsc.VectorSubcoreMesh(core_axis_name="c", subcore_axis_name="s")
    k = pl.kernel(
        _gather_body,
        out_type=jax.ShapeDtypeStruct((3, 2, 128, 8, 128), jnp.float32),
        mesh=mesh,
        scratch_types=[
            pltpu.VMEM((512, 128), jnp.float32),
            pltpu.VMEM((128, 128), jnp.int32),
            pltpu.VMEM((128, 128), jnp.float32),
        ],
        compiler_params=pltpu.CompilerParams(
            use_tc_tiling_on_sc=False, needs_layout_passes=False
        ),
    )
    out5 = k(t5, idx5)
    return out5.transpose(1, 3, 2, 4, 0).reshape(16, 16384, 3)


# speedup vs baseline: 1.0362x; 1.0362x over previous
import jax
import jax.numpy as jnp
from jax import lax
from jax.experimental import pallas as pl
from jax.experimental.pallas import tpu as pltpu
from jax.experimental.pallas import tpu_sc as plsc


def _gather_body(t5, idx5, out5, plane_v, idx_v, out_v, sem_i, sem_p, sem_o):
    c = lax.axis_index("c")
    s = lax.axis_index("s")
    wid = s * 2 + c
    heavy = wid < 16
    b = lax.select(heavy, wid, wid - 16)
    bt = b // 8
    rb = b % 8
    p1 = lax.select(heavy, 0, 1)

    pltpu.async_copy(idx5.at[bt, :, rb, :], idx_v, sem_i)
    pltpu.async_copy(t5.at[p1, bt, :, rb, :], plane_v, sem_p)
    pltpu.make_async_copy(idx5.at[bt, :, rb, :], idx_v, sem_i).wait()
    pltpu.make_async_copy(t5.at[p1, bt, :, rb, :], plane_v, sem_p).wait()

    def gather():
        @plsc.parallel_loop(0, 1024, step=1, unroll=8)
        def _(k):
            r = lax.shift_right_logical(k, 3)
            o = lax.bitwise_and(k, 7) * 16
            n = idx_v[r, pl.ds(o, 16)]
            hi = lax.shift_right_logical(n, 7)
            lo = lax.bitwise_and(n, 127)
            out_v[r, pl.ds(o, 16)] = plsc.load_gather(plane_v, [hi, lo])

    gather()

    @pl.when(heavy)
    def _():
        pltpu.async_copy(out_v, out5.at[0, bt, :, rb, :], sem_o)
        pltpu.async_copy(t5.at[2, bt, :, rb, :], plane_v, sem_p)
        pltpu.make_async_copy(out_v, out5.at[0, bt, :, rb, :], sem_o).wait()
        pltpu.make_async_copy(t5.at[2, bt, :, rb, :], plane_v, sem_p).wait()
        gather()
        pltpu.sync_copy(out_v, out5.at[2, bt, :, rb, :])

    @pl.when(jnp.logical_not(heavy))
    def _():
        pltpu.sync_copy(out_v, out5.at[1, bt, :, rb, :])


def kernel(t_in, t_idx):
    t5 = t_in.transpose(2, 0, 1).reshape(3, 2, 8, 512, 128).transpose(0, 1, 3, 2, 4)
    idx5 = t_idx.astype(jnp.int32).reshape(2, 8, 128, 128).transpose(0, 2, 1, 3)
    mesh = plsc.VectorSubcoreMesh(core_axis_name="c", subcore_axis_name="s")
    k = pl.kernel(
        _gather_body,
        out_type=jax.ShapeDtypeStruct((3, 2, 128, 8, 128), jnp.float32),
        mesh=mesh,
        scratch_types=[
            pltpu.VMEM((512, 128), jnp.float32),
            pltpu.VMEM((128, 128), jnp.int32),
            pltpu.VMEM((128, 128), jnp.float32),
            pltpu.SemaphoreType.DMA,
            pltpu.SemaphoreType.DMA,
            pltpu.SemaphoreType.DMA,
        ],
        compiler_params=pltpu.CompilerParams(
            use_tc_tiling_on_sc=False, needs_layout_passes=False
        ),
    )
    out5 = k(t5, idx5)
    return out5.transpose(1, 3, 2, 4, 0).reshape(16, 16384, 3)
